# baseline (device time: 9410 ns/iter reference)
import jax
import jax.numpy as jnp
from jax import lax
from jax.experimental import pallas as pl
from jax.experimental.pallas import tpu as pltpu


def kernel(x):
    _, m, n_half = x.shape
    n = 2 * n_half

    def body(x_ref, out_ref, send_ref, recv_x_ref, recv_y_ref, recv_d_ref,
             send_sems, recv_sems):
        my_x = lax.axis_index("x")
        my_y = lax.axis_index("y")
        peers = [
            (1 - my_x, my_y),
            (my_x, 1 - my_y),
            (1 - my_x, 1 - my_y),
        ]

        barrier_sem = pltpu.get_barrier_semaphore()
        for p in peers:
            pl.semaphore_signal(barrier_sem, inc=1, device_id=p,
                                device_id_type=pl.DeviceIdType.MESH)

        send_ref[:, :] = x_ref[0].astype(jnp.bfloat16)
        pl.semaphore_wait(barrier_sem, 3)

        m_half = m // 2
        rdma_d = pltpu.make_async_remote_copy(
            src_ref=send_ref,
            dst_ref=recv_d_ref,
            send_sem=send_sems.at[0],
            recv_sem=recv_sems.at[0],
            device_id=peers[2],
            device_id_type=pl.DeviceIdType.MESH,
        )
        rdma_d.start()
        rdma_y = pltpu.make_async_remote_copy(
            src_ref=send_ref,
            dst_ref=recv_y_ref,
            send_sem=send_sems.at[1],
            recv_sem=recv_sems.at[1],
            device_id=peers[1],
            device_id_type=pl.DeviceIdType.MESH,
        )
        rdma_y.start()
        rdma_x0 = pltpu.make_async_remote_copy(
            src_ref=send_ref.at[pl.ds(0, m_half)],
            dst_ref=recv_x_ref.at[pl.ds(0, m_half)],
            send_sem=send_sems.at[2],
            recv_sem=recv_sems.at[2],
            device_id=peers[0],
            device_id_type=pl.DeviceIdType.MESH,
        )
        rdma_x0.start()
        rdma_x1 = pltpu.make_async_remote_copy(
            src_ref=send_ref.at[pl.ds(m_half, m_half)],
            dst_ref=recv_x_ref.at[pl.ds(m_half, m_half)],
            send_sem=send_sems.at[3],
            recv_sem=recv_sems.at[3],
            device_id=peers[0],
            device_id_type=pl.DeviceIdType.MESH,
        )
        rdma_x1.start()

        col = my_y * n_half
        other = (1 - my_y) * n_half

        rdma_x0.wait_recv()
        out_ref[pl.ds(0, m_half), pl.ds(col, n_half)] = (
            send_ref[pl.ds(0, m_half), :] + recv_x_ref[pl.ds(0, m_half), :]
        )
        rdma_x1.wait_recv()
        out_ref[pl.ds(m_half, m_half), pl.ds(col, n_half)] = (
            send_ref[pl.ds(m_half, m_half), :]
            + recv_x_ref[pl.ds(m_half, m_half), :]
        )

        rdma_y.wait_recv()
        rdma_d.wait_recv()
        out_ref[:, pl.ds(other, n_half)] = recv_y_ref[:, :] + recv_d_ref[:, :]

        for r in (rdma_d, rdma_y, rdma_x0, rdma_x1):
            r.wait_send()

    return pl.pallas_call(
        body,
        out_shape=jax.ShapeDtypeStruct((m, n), jnp.bfloat16),
        in_specs=[pl.BlockSpec(memory_space=pltpu.VMEM)],
        out_specs=pl.BlockSpec(memory_space=pltpu.VMEM),
        scratch_shapes=[
            pltpu.VMEM((m, n_half), jnp.bfloat16),
            pltpu.VMEM((m, n_half), jnp.bfloat16),
            pltpu.VMEM((m, n_half), jnp.bfloat16),
            pltpu.VMEM((m, n_half), jnp.bfloat16),
            pltpu.SemaphoreType.DMA((4,)),
            pltpu.SemaphoreType.DMA((4,)),
        ],
        compiler_params=pltpu.CompilerParams(collective_id=0),
    )(x)


# device time: 9111 ns/iter; 1.0328x vs baseline; 1.0328x over previous
import jax
import jax.numpy as jnp
from jax import lax
from jax.experimental import pallas as pl
from jax.experimental.pallas import tpu as pltpu


def kernel(x):
    _, m, n_half = x.shape
    n = 2 * n_half

    def body(x_ref, out_ref, send_ref, recv_x_ref, recv_y_ref, recv_d_ref,
             send_sems, recv_sems):
        my_x = lax.axis_index("x")
        my_y = lax.axis_index("y")
        peers = [
            (1 - my_x, my_y),
            (my_x, 1 - my_y),
            (1 - my_x, 1 - my_y),
        ]

        barrier_sem = pltpu.get_barrier_semaphore()
        for p in peers:
            pl.semaphore_signal(barrier_sem, inc=1, device_id=p,
                                device_id_type=pl.DeviceIdType.MESH)

        send_ref[:, :] = x_ref[0].astype(jnp.bfloat16)
        pl.semaphore_wait(barrier_sem, 3)

        rdmas = []
        for i, (p, dst) in enumerate(
            zip(peers, [recv_x_ref, recv_y_ref, recv_d_ref])
        ):
            rdma = pltpu.make_async_remote_copy(
                src_ref=send_ref,
                dst_ref=dst,
                send_sem=send_sems.at[i],
                recv_sem=recv_sems.at[i],
                device_id=p,
                device_id_type=pl.DeviceIdType.MESH,
            )
            rdma.start()
            rdmas.append(rdma)

        col = my_y * n_half
        other = (1 - my_y) * n_half

        rdmas[0].wait_recv()
        out_ref[:, pl.ds(col, n_half)] = send_ref[:, :] + recv_x_ref[:, :]

        rdmas[1].wait_recv()
        rdmas[2].wait_recv()
        out_ref[:, pl.ds(other, n_half)] = recv_y_ref[:, :] + recv_d_ref[:, :]

        for r in rdmas:
            r.wait_send()

    return pl.pallas_call(
        body,
        out_shape=jax.ShapeDtypeStruct((m, n), jnp.bfloat16),
        in_specs=[pl.BlockSpec(memory_space=pltpu.VMEM)],
        out_specs=pl.BlockSpec(memory_space=pltpu.VMEM),
        scratch_shapes=[
            pltpu.VMEM((m, n_half), jnp.bfloat16),
            pltpu.VMEM((m, n_half), jnp.bfloat16),
            pltpu.VMEM((m, n_half), jnp.bfloat16),
            pltpu.VMEM((m, n_half), jnp.bfloat16),
            pltpu.SemaphoreType.DMA((3,)),
            pltpu.SemaphoreType.DMA((3,)),
        ],
        compiler_params=pltpu.CompilerParams(collective_id=0),
    )(x)
